# baseline (speedup 1.0000x reference)
import jax
import jax.numpy as jnp
from jax import lax
from jax.experimental import pallas as pl
from jax.experimental.pallas import tpu as pltpu
from jax.experimental.pallas import tpu_sc as plsc

_B = 64
_S = 2048
_D = 1024
_L = 16
_CHUNK = 128
_KPB = _D // _CHUNK


def _body(table_hbm, tl_hbm, out_hbm, tl_v, idx_v, oidx_v, rows_v, sem):
    wid = lax.axis_index("s") * 2 + lax.axis_index("c")
    g = wid // _KPB
    k = wid % _KPB

    pltpu.sync_copy(table_hbm.at[pl.ds(wid * _L, _L)], rows_v)
    pltpu.sync_copy(rows_v, out_hbm.at[pl.ds(wid * _L, _L)])


@jax.jit
def kernel(sequence_embedding, token_len):
    table = sequence_embedding.reshape(_B * _S * _KPB, _CHUNK)
    mesh = plsc.VectorSubcoreMesh(core_axis_name="c", subcore_axis_name="s", num_cores=1)
    out = pl.kernel(
        _body,
        out_type=jax.ShapeDtypeStruct((_B * _KPB, _CHUNK), jnp.float32),
        mesh=mesh,
        scratch_types=[
            pltpu.VMEM((_L,), jnp.int32),
            pltpu.VMEM((_L,), jnp.int32),
            pltpu.VMEM((_L,), jnp.int32),
            pltpu.VMEM((_L, _CHUNK), jnp.float32),
            pltpu.SemaphoreType.DMA,
        ],
    )(table, token_len)
    return out.reshape(_B, _D)


# speedup vs baseline: 21.3021x; 21.3021x over previous
import jax
import jax.numpy as jnp
from jax import lax
from jax.experimental import pallas as pl
from jax.experimental.pallas import tpu as pltpu
from jax.experimental.pallas import tpu_sc as plsc

_B = 64
_S = 2048
_D = 1024


def _body(seq_hbm, tl_hbm, out_hbm, tl_s, sem):
    pltpu.sync_copy(tl_hbm, tl_s)
    copies = []
    for b in range(_B):
        t = tl_s[b]
        row = jnp.where(t == 0, _S - 1, t - 1)
        c = pltpu.make_async_copy(seq_hbm.at[b, row], out_hbm.at[b], sem)
        c.start()
        copies.append(c)
    for c in copies:
        c.wait()


@jax.jit
def kernel(sequence_embedding, token_len):
    mesh = plsc.ScalarSubcoreMesh(axis_name="c", num_cores=1)
    out = pl.kernel(
        _body,
        out_type=jax.ShapeDtypeStruct((_B, _D), jnp.float32),
        mesh=mesh,
        scratch_types=[
            pltpu.SMEM((_B,), jnp.int32),
            pltpu.SemaphoreType.DMA,
        ],
    )(sequence_embedding, token_len)
    return out
